# baseline (speedup 1.0000x reference)
import functools

import jax
import jax.numpy as jnp
from jax import lax
from jax.experimental import pallas as pl
from jax.experimental.pallas import tpu as pltpu
from jax.experimental.pallas import tpu_sc as plsc

_NCORES = 2
_NSUB = 16


def _elu(v):
    return jnp.where(v > 0, v, jnp.exp(v) - 1.0)


def _sc_gather(table, idx):
    np_, d = table.shape
    e = idx.shape[0]
    nw = _NCORES * _NSUB
    bpw = e // nw

    mesh = plsc.VectorSubcoreMesh(core_axis_name="c", subcore_axis_name="s")

    @functools.partial(
        pl.kernel,
        out_type=jax.ShapeDtypeStruct((e, d), jnp.float32),
        mesh=mesh,
        scratch_types=[
            pltpu.VMEM((bpw,), jnp.int32),
            pltpu.VMEM((bpw, d), jnp.float32),
            pltpu.SemaphoreType.DMA,
        ],
    )
    def k(table_hbm, idx_hbm, out_hbm, idx_v, rows_v, sem):
        wid = lax.axis_index("s") * _NCORES + lax.axis_index("c")
        base = wid * bpw
        pltpu.sync_copy(idx_hbm.at[pl.ds(base, bpw)], idx_v)
        pltpu.async_copy(table_hbm.at[idx_v], rows_v, sem).wait()
        pltpu.sync_copy(rows_v, out_hbm.at[pl.ds(base, bpw)])

    return k(table, idx)


_COLW = 8


_ECH = 128


def _sc_scatter_cols(msg, zeros_flat, dst1d):
    e, out = msg.shape
    na = out // _COLW
    nf = zeros_flat.shape[0]

    mesh = plsc.VectorSubcoreMesh(core_axis_name="c", subcore_axis_name="s")

    @functools.partial(
        pl.kernel,
        out_type=jax.ShapeDtypeStruct((na, nf, 128), jnp.float32),
        mesh=mesh,
        compiler_params=pltpu.CompilerParams(needs_layout_passes=False),
        scratch_types=[
            pltpu.VMEM((e,), jnp.int32),
            pltpu.VMEM((_ECH, 128), jnp.float32),
            pltpu.VMEM((_ECH, 128), jnp.float32),
            pltpu.VMEM((nf, 128), jnp.float32),
            pltpu.SemaphoreType.DMA,
            pltpu.SemaphoreType.DMA,
        ],
    )
    def k(msg_hbm, zeros_hbm, dst_hbm, agg_hbm, idx_v, chunk_a, chunk_b,
          acc, sem_a, sem_b):
        w = lax.axis_index("s") * _NCORES + lax.axis_index("c")

        @pl.when(w < na)
        def _():
            pltpu.sync_copy(zeros_hbm, acc)
            pltpu.sync_copy(dst_hbm, idx_v)
            lanes = lax.iota(jnp.int32, 16)
            lo = lanes < _COLW
            half = (w // _NSUB) * 128
            wcol = w * _COLW - half
            colsel = wcol + (lanes & (_COLW - 1))
            bufs = ((chunk_a, sem_a), (chunk_b, sem_b))
            nch = e // _ECH

            def start(ch):
                buf, sem = bufs[ch % 2]
                return pltpu.async_copy(
                    msg_hbm.at[pl.ds(ch * _ECH, _ECH), pl.ds(half, 128)],
                    buf, sem)

            cp = start(0)
            for ch in range(nch):
                cp.wait()
                if ch + 1 < nch:
                    cp = start(ch + 1)
                chunk_v = bufs[ch % 2][0]

                def body(i, _):
                    base = 8 * i
                    loaded = []
                    for u in range(4):
                        lepair = base + 2 * u + (lanes >> 3)
                        rows = plsc.load_gather(idx_v, [ch * _ECH + lepair])
                        vals = plsc.load_gather(chunk_v, [lepair, colsel])
                        aflat = rows * _COLW + (lanes & (_COLW - 1))
                        loaded.append((aflat >> 7, aflat & 127, vals))
                    for ar, al, vals in loaded:
                        plsc.addupdate_scatter(acc, [ar, al], vals, mask=lo)
                        plsc.addupdate_scatter(acc, [ar, al], vals, mask=~lo)
                    return 0

                lax.fori_loop(0, _ECH // 8, body, 0)
            pltpu.sync_copy(acc, agg_hbm.at[w])

    return k(msg, zeros_flat, dst1d)


def _tc_messages(xg, eattr, w1, b1, w2r, b2r, *, ci, sub, te, apply_elu,
                 lci=None):
    e, in_pad = xg.shape
    hid = w1.shape[1]
    out = w2r.shape[1]
    if lci is None:
        lci = ci
    kc = lci * hid
    ic_n = w2r.shape[0] // kc
    ea_d = eattr.shape[1]

    def body(eattr_ref, xg_ref, w1_ref, b1_ref, w2r_ref, b2r_ref, acc_ref):
        ic = pl.program_id(0)
        et_i = pl.program_id(1)
        h = jnp.maximum(
            jnp.dot(eattr_ref[...], w1_ref[...],
                    preferred_element_type=jnp.float32) + b1_ref[...], 0.0)
        xc = xg_ref[...][:, :lci]
        if apply_elu:
            xc = _elu(xc)
        contrib = jnp.dot(xc, b2r_ref[...], preferred_element_type=jnp.float32)
        hb = h.astype(jnp.bfloat16)
        xb = xc.astype(jnp.bfloat16)
        for i in range(lci):
            pb = xb[:, i:i + 1] * hb
            contrib = contrib + jnp.dot(
                pb, w2r_ref[i * hid:(i + 1) * hid, :],
                preferred_element_type=jnp.float32)
        rows = pl.ds(et_i * te, te)

        @pl.when(ic == 0)
        def _():
            acc_ref[rows, :] = contrib

        @pl.when(ic != 0)
        def _():
            acc_ref[rows, :] = acc_ref[rows, :] + contrib

    return pl.pallas_call(
        body,
        grid=(ic_n, e // te),
        in_specs=[
            pl.BlockSpec((te, ea_d), lambda ic, et_i: (et_i, 0)),
            pl.BlockSpec((te, ci), lambda ic, et_i: (et_i, ic)),
            pl.BlockSpec(w1.shape, lambda ic, et_i: (0, 0)),
            pl.BlockSpec((1, hid), lambda ic, et_i: (0, 0)),
            pl.BlockSpec((kc, out), lambda ic, et_i: (ic, 0)),
            pl.BlockSpec((lci, out), lambda ic, et_i: (ic, 0)),
        ],
        out_specs=pl.BlockSpec((e, out), lambda ic, et_i: (0, 0)),
        out_shape=jax.ShapeDtypeStruct((e, out), jnp.float32),
    )(eattr, xg, w1, b1.reshape(1, hid), w2r, b2r)


def _tc_combine(agg_flat, x, rootw, bias, *, tn, apply_elu):
    nw, n, colw = agg_flat.shape
    cin = x.shape[1]
    out = rootw.shape[1]

    def body(a_ref, x_ref, w_ref, b_ref, o_ref):
        xt = x_ref[...]
        if apply_elu:
            xt = _elu(xt)
        agg = jnp.concatenate([a_ref[i] for i in range(nw)], axis=1)
        o_ref[...] = agg + jnp.dot(xt, w_ref[...],
                                   preferred_element_type=jnp.float32) + b_ref[...]

    return pl.pallas_call(
        body,
        grid=(n // tn,),
        in_specs=[
            pl.BlockSpec((nw, tn, colw), lambda i: (0, i, 0)),
            pl.BlockSpec((tn, cin), lambda i: (i, 0)),
            pl.BlockSpec(rootw.shape, lambda i: (0, 0)),
            pl.BlockSpec((1, out), lambda i: (0, 0)),
        ],
        out_specs=pl.BlockSpec((tn, out), lambda i: (i, 0)),
        out_shape=jax.ShapeDtypeStruct((n, out), jnp.float32),
    )(agg_flat, x, rootw, bias.reshape(1, out))


def _tc_head(z, fc1_w, fc1_b, fc2_w, fc2_b, fc3_w, fc3_b, *, tn):
    n, d = z.shape

    def body(z_ref, w1_ref, b1_ref, w2_ref, b2_ref, w3_ref, b3_ref, o_ref):
        h = _elu(z_ref[...])
        h = _elu(jnp.dot(h, w1_ref[...],
                         preferred_element_type=jnp.float32) + b1_ref[...])
        h = _elu(jnp.dot(h, w2_ref[...],
                         preferred_element_type=jnp.float32) + b2_ref[...])
        o_ref[...] = jnp.dot(h, w3_ref[...],
                             preferred_element_type=jnp.float32) + b3_ref[...]

    return pl.pallas_call(
        body,
        grid=(n // tn,),
        in_specs=[
            pl.BlockSpec((tn, d), lambda i: (i, 0)),
            pl.BlockSpec(fc1_w.shape, lambda i: (0, 0)),
            pl.BlockSpec((1, fc1_w.shape[1]), lambda i: (0, 0)),
            pl.BlockSpec(fc2_w.shape, lambda i: (0, 0)),
            pl.BlockSpec((1, fc2_w.shape[1]), lambda i: (0, 0)),
            pl.BlockSpec(fc3_w.shape, lambda i: (0, 0)),
            pl.BlockSpec((1, fc3_w.shape[1]), lambda i: (0, 0)),
        ],
        out_specs=pl.BlockSpec((tn, 1), lambda i: (i, 0)),
        out_shape=jax.ShapeDtypeStruct((n, 1), jnp.float32),
    )(z, fc1_w, fc1_b.reshape(1, -1), fc2_w, fc2_b.reshape(1, -1),
      fc3_w, fc3_b.reshape(1, -1))


def _prep_w2(w2, b2, hid, cin, cout, in_pad):
    w2r = w2.reshape(hid, cin, cout).transpose(1, 0, 2)
    w2r = jnp.pad(w2r, ((0, in_pad - cin), (0, 0), (0, 0)))
    b2r = jnp.pad(b2.reshape(cin, cout), ((0, in_pad - cin), (0, 0)))
    return w2r.reshape(in_pad * hid, cout).astype(jnp.bfloat16), b2r


def kernel(x, edge_index, edge_attr,
           c1_w1, c1_b1, c1_w2, c1_b2, c1_root, c1_bias,
           c2_w1, c2_b1, c2_w2, c2_b2, c2_root, c2_bias,
           c3_w1, c3_b1, c3_w2, c3_b2, c3_root, c3_bias,
           fc1_w, fc1_b, fc2_w, fc2_b, fc3_w, fc3_b):
    e = edge_index.shape[1]
    src = edge_index[0]
    dst1d = edge_index[1]
    n = x.shape[0]
    zeros_flat = jnp.zeros((n * _COLW // 128, 128), jnp.float32)


    in1 = x.shape[1]
    in1_pad = 48
    x_pad = jnp.pad(x, ((0, 0), (0, 128 - in1)))
    w2r1, b2r1 = _prep_w2(c1_w2, c1_b2, 512, in1, 128, in1_pad)
    root1 = jnp.pad(c1_root, ((0, 128 - in1), (0, 0)))

    xg1 = _sc_gather(x_pad, src)
    msg1 = _tc_messages(xg1, edge_attr, c1_w1, c1_b1, w2r1, b2r1,
                        ci=128, lci=48, sub=8, te=256, apply_elu=False)
    agg1 = _sc_scatter_cols(msg1, zeros_flat, dst1d)
    z1 = _tc_combine(agg1.reshape(-1, n, _COLW), x_pad, root1, c1_bias,
                     tn=1000, apply_elu=False)

    w2r2, b2r2 = _prep_w2(c2_w2, c2_b2, 128, 128, 256, 128)
    xg2 = _sc_gather(z1, src)
    msg2 = _tc_messages(xg2, edge_attr, c2_w1, c2_b1, w2r2, b2r2,
                        ci=128, sub=32, te=256, apply_elu=True)
    agg2 = _sc_scatter_cols(msg2, zeros_flat, dst1d)
    z2 = _tc_combine(agg2.reshape(-1, n, _COLW), z1, c2_root, c2_bias,
                     tn=1000, apply_elu=True)

    w2r3, b2r3 = _prep_w2(c3_w2, c3_b2, 128, 256, 256, 256)
    xg3 = _sc_gather(z2, src)
    msg3 = _tc_messages(xg3, edge_attr, c3_w1, c3_b1, w2r3, b2r3,
                        ci=128, sub=32, te=256, apply_elu=True)
    agg3 = _sc_scatter_cols(msg3, zeros_flat, dst1d)
    z3 = _tc_combine(agg3.reshape(-1, n, _COLW), z2, c3_root, c3_bias,
                     tn=1000, apply_elu=True)

    y = _tc_head(z3, fc1_w, fc1_b, fc2_w, fc2_b, fc3_w, fc3_b, tn=1000)
    return y.reshape(-1)


# speedup vs baseline: 1.0103x; 1.0103x over previous
import functools

import jax
import jax.numpy as jnp
from jax import lax
from jax.experimental import pallas as pl
from jax.experimental.pallas import tpu as pltpu
from jax.experimental.pallas import tpu_sc as plsc

_NCORES = 2
_NSUB = 16


def _elu(v):
    return jnp.where(v > 0, v, jnp.exp(v) - 1.0)


def _sc_gather(table, idx):
    np_, d = table.shape
    e = idx.shape[0]
    nw = _NCORES * _NSUB
    bpw = e // nw

    mesh = plsc.VectorSubcoreMesh(core_axis_name="c", subcore_axis_name="s")

    @functools.partial(
        pl.kernel,
        out_type=jax.ShapeDtypeStruct((e, d), jnp.float32),
        mesh=mesh,
        scratch_types=[
            pltpu.VMEM((bpw,), jnp.int32),
            pltpu.VMEM((bpw, d), jnp.float32),
            pltpu.SemaphoreType.DMA,
        ],
    )
    def k(table_hbm, idx_hbm, out_hbm, idx_v, rows_v, sem):
        wid = lax.axis_index("s") * _NCORES + lax.axis_index("c")
        base = wid * bpw
        pltpu.sync_copy(idx_hbm.at[pl.ds(base, bpw)], idx_v)
        pltpu.async_copy(table_hbm.at[idx_v], rows_v, sem).wait()
        pltpu.sync_copy(rows_v, out_hbm.at[pl.ds(base, bpw)])

    return k(table, idx)


_COLW = 8


_ECH = 128


def _sc_scatter_cols(msg, zeros_flat, dst1d):
    e, out = msg.shape
    na = out // _COLW
    nf = zeros_flat.shape[0]

    mesh = plsc.VectorSubcoreMesh(core_axis_name="c", subcore_axis_name="s")

    @functools.partial(
        pl.kernel,
        out_type=jax.ShapeDtypeStruct((na, nf, 128), jnp.float32),
        mesh=mesh,
        compiler_params=pltpu.CompilerParams(needs_layout_passes=False),
        scratch_types=[
            pltpu.VMEM((e,), jnp.int32),
            pltpu.VMEM((_ECH, 128), jnp.float32),
            pltpu.VMEM((_ECH, 128), jnp.float32),
            pltpu.VMEM((nf, 128), jnp.float32),
            pltpu.SemaphoreType.DMA,
            pltpu.SemaphoreType.DMA,
        ],
    )
    def k(msg_hbm, zeros_hbm, dst_hbm, agg_hbm, idx_v, chunk_a, chunk_b,
          acc, sem_a, sem_b):
        w = lax.axis_index("s") * _NCORES + lax.axis_index("c")

        @pl.when(w < na)
        def _():
            pltpu.sync_copy(zeros_hbm, acc)
            pltpu.sync_copy(dst_hbm, idx_v)
            lanes = lax.iota(jnp.int32, 16)
            lo = lanes < _COLW
            half = (w // _NSUB) * 128
            wcol = w * _COLW - half
            colsel = wcol + (lanes & (_COLW - 1))
            bufs = ((chunk_a, sem_a), (chunk_b, sem_b))
            nch = e // _ECH

            def start(ch):
                buf, sem = bufs[ch % 2]
                return pltpu.async_copy(
                    msg_hbm.at[pl.ds(ch * _ECH, _ECH), pl.ds(half, 128)],
                    buf, sem)

            cp = start(0)
            for ch in range(nch):
                cp.wait()
                if ch + 1 < nch:
                    cp = start(ch + 1)
                chunk_v = bufs[ch % 2][0]

                def body(i, _):
                    base = 8 * i
                    loaded = []
                    for u in range(4):
                        lepair = base + 2 * u + (lanes >> 3)
                        rows = plsc.load_gather(idx_v, [ch * _ECH + lepair])
                        vals = plsc.load_gather(chunk_v, [lepair, colsel])
                        aflat = rows * _COLW + (lanes & (_COLW - 1))
                        loaded.append((aflat >> 7, aflat & 127, vals))
                    for ar, al, vals in loaded:
                        plsc.addupdate_scatter(acc, [ar, al], vals, mask=lo)
                        plsc.addupdate_scatter(acc, [ar, al], vals, mask=~lo)
                    return 0

                lax.fori_loop(0, _ECH // 8, body, 0)
            pltpu.sync_copy(acc, agg_hbm.at[w])

    return k(msg, zeros_flat, dst1d)


def _tc_messages(xg, eattr, w1, b1, w2r, b2r, *, ci, sub, te, apply_elu,
                 lci=None):
    e, in_pad = xg.shape
    hid = w1.shape[1]
    out = w2r.shape[1]
    if lci is None:
        lci = ci
    kc = lci * hid
    ic_n = w2r.shape[0] // kc
    ea_d = eattr.shape[1]

    def body(eattr_ref, xg_ref, w1_ref, b1_ref, w2r_ref, b2r_ref, acc_ref):
        ic = pl.program_id(0)
        et_i = pl.program_id(1)
        h = jnp.maximum(
            jnp.dot(eattr_ref[...], w1_ref[...],
                    preferred_element_type=jnp.float32) + b1_ref[...], 0.0)
        xc = xg_ref[...][:, :lci]
        if apply_elu:
            xc = _elu(xc)
        contrib = jnp.dot(xc, b2r_ref[...], preferred_element_type=jnp.float32)
        hb = h.astype(jnp.bfloat16)
        xb = xc.astype(jnp.bfloat16)
        for i in range(lci):
            pb = xb[:, i:i + 1] * hb
            contrib = contrib + jnp.dot(
                pb, w2r_ref[i * hid:(i + 1) * hid, :],
                preferred_element_type=jnp.float32)
        rows = pl.ds(et_i * te, te)

        @pl.when(ic == 0)
        def _():
            acc_ref[rows, :] = contrib

        @pl.when(ic != 0)
        def _():
            acc_ref[rows, :] = acc_ref[rows, :] + contrib

    return pl.pallas_call(
        body,
        grid=(ic_n, e // te),
        in_specs=[
            pl.BlockSpec((te, ea_d), lambda ic, et_i: (et_i, 0)),
            pl.BlockSpec((te, ci), lambda ic, et_i: (et_i, ic)),
            pl.BlockSpec(w1.shape, lambda ic, et_i: (0, 0)),
            pl.BlockSpec((1, hid), lambda ic, et_i: (0, 0)),
            pl.BlockSpec((kc, out), lambda ic, et_i: (ic, 0)),
            pl.BlockSpec((lci, out), lambda ic, et_i: (ic, 0)),
        ],
        out_specs=pl.BlockSpec((e, out), lambda ic, et_i: (0, 0)),
        out_shape=jax.ShapeDtypeStruct((e, out), jnp.float32),
    )(eattr, xg, w1, b1.reshape(1, hid), w2r, b2r)


def _tc_combine(agg_flat, x, rootw, bias, *, tn, apply_elu):
    nw, n, colw = agg_flat.shape
    cin = x.shape[1]
    out = rootw.shape[1]

    def body(a_ref, x_ref, w_ref, b_ref, o_ref):
        xt = x_ref[...]
        if apply_elu:
            xt = _elu(xt)
        agg = jnp.concatenate([a_ref[i] for i in range(nw)], axis=1)
        o_ref[...] = agg + jnp.dot(xt, w_ref[...],
                                   preferred_element_type=jnp.float32) + b_ref[...]

    return pl.pallas_call(
        body,
        grid=(n // tn,),
        in_specs=[
            pl.BlockSpec((nw, tn, colw), lambda i: (0, i, 0)),
            pl.BlockSpec((tn, cin), lambda i: (i, 0)),
            pl.BlockSpec(rootw.shape, lambda i: (0, 0)),
            pl.BlockSpec((1, out), lambda i: (0, 0)),
        ],
        out_specs=pl.BlockSpec((tn, out), lambda i: (i, 0)),
        out_shape=jax.ShapeDtypeStruct((n, out), jnp.float32),
    )(agg_flat, x, rootw, bias.reshape(1, out))


def _tc_head(agg_sh, x, rootw, rbias, fc1_w, fc1_b, fc2_w, fc2_b,
             fc3_w, fc3_b, *, tn):
    nw, n, colw = agg_sh.shape
    cin = x.shape[1]
    out = rootw.shape[1]

    def body(a_ref, x_ref, rw_ref, rb_ref, w1_ref, b1_ref, w2_ref, b2_ref,
             w3_ref, b3_ref, o_ref):
        agg = jnp.concatenate([a_ref[i] for i in range(nw)], axis=1)
        z = agg + jnp.dot(_elu(x_ref[...]), rw_ref[...],
                          preferred_element_type=jnp.float32) + rb_ref[...]
        h = _elu(z)
        h = _elu(jnp.dot(h, w1_ref[...],
                         preferred_element_type=jnp.float32) + b1_ref[...])
        h = _elu(jnp.dot(h, w2_ref[...],
                         preferred_element_type=jnp.float32) + b2_ref[...])
        o_ref[...] = jnp.dot(h, w3_ref[...],
                             preferred_element_type=jnp.float32) + b3_ref[...]

    return pl.pallas_call(
        body,
        grid=(n // tn,),
        in_specs=[
            pl.BlockSpec((nw, tn, colw), lambda i: (0, i, 0)),
            pl.BlockSpec((tn, cin), lambda i: (i, 0)),
            pl.BlockSpec(rootw.shape, lambda i: (0, 0)),
            pl.BlockSpec((1, out), lambda i: (0, 0)),
            pl.BlockSpec(fc1_w.shape, lambda i: (0, 0)),
            pl.BlockSpec((1, fc1_w.shape[1]), lambda i: (0, 0)),
            pl.BlockSpec(fc2_w.shape, lambda i: (0, 0)),
            pl.BlockSpec((1, fc2_w.shape[1]), lambda i: (0, 0)),
            pl.BlockSpec(fc3_w.shape, lambda i: (0, 0)),
            pl.BlockSpec((1, fc3_w.shape[1]), lambda i: (0, 0)),
        ],
        out_specs=pl.BlockSpec((tn, 1), lambda i: (i, 0)),
        out_shape=jax.ShapeDtypeStruct((n, 1), jnp.float32),
    )(agg_sh, x, rootw, rbias.reshape(1, out), fc1_w, fc1_b.reshape(1, -1),
      fc2_w, fc2_b.reshape(1, -1), fc3_w, fc3_b.reshape(1, -1))


def _prep_w2(w2, b2, hid, cin, cout, in_pad):
    w2r = w2.reshape(hid, cin, cout).transpose(1, 0, 2)
    w2r = jnp.pad(w2r, ((0, in_pad - cin), (0, 0), (0, 0)))
    b2r = jnp.pad(b2.reshape(cin, cout), ((0, in_pad - cin), (0, 0)))
    return w2r.reshape(in_pad * hid, cout).astype(jnp.bfloat16), b2r


def kernel(x, edge_index, edge_attr,
           c1_w1, c1_b1, c1_w2, c1_b2, c1_root, c1_bias,
           c2_w1, c2_b1, c2_w2, c2_b2, c2_root, c2_bias,
           c3_w1, c3_b1, c3_w2, c3_b2, c3_root, c3_bias,
           fc1_w, fc1_b, fc2_w, fc2_b, fc3_w, fc3_b):
    e = edge_index.shape[1]
    src = edge_index[0]
    dst1d = edge_index[1]
    n = x.shape[0]
    zeros_flat = jnp.zeros((n * _COLW // 128, 128), jnp.float32)


    in1 = x.shape[1]
    in1_pad = 48
    x_pad = jnp.pad(x, ((0, 0), (0, 128 - in1)))
    w2r1, b2r1 = _prep_w2(c1_w2, c1_b2, 512, in1, 128, in1_pad)
    root1 = jnp.pad(c1_root, ((0, 128 - in1), (0, 0)))

    xg1 = _sc_gather(x_pad, src)
    msg1 = _tc_messages(xg1, edge_attr, c1_w1, c1_b1, w2r1, b2r1,
                        ci=128, lci=48, sub=8, te=256, apply_elu=False)
    agg1 = _sc_scatter_cols(msg1, zeros_flat, dst1d)
    z1 = _tc_combine(agg1.reshape(-1, n, _COLW), x_pad, root1, c1_bias,
                     tn=1000, apply_elu=False)

    w2r2, b2r2 = _prep_w2(c2_w2, c2_b2, 128, 128, 256, 128)
    xg2 = _sc_gather(z1, src)
    msg2 = _tc_messages(xg2, edge_attr, c2_w1, c2_b1, w2r2, b2r2,
                        ci=128, sub=32, te=256, apply_elu=True)
    agg2 = _sc_scatter_cols(msg2, zeros_flat, dst1d)
    z2 = _tc_combine(agg2.reshape(-1, n, _COLW), z1, c2_root, c2_bias,
                     tn=1000, apply_elu=True)

    w2r3, b2r3 = _prep_w2(c3_w2, c3_b2, 128, 256, 256, 256)
    xg3 = _sc_gather(z2, src)
    msg3 = _tc_messages(xg3, edge_attr, c3_w1, c3_b1, w2r3, b2r3,
                        ci=128, sub=32, te=256, apply_elu=True)
    agg3 = _sc_scatter_cols(msg3, zeros_flat, dst1d)

    y = _tc_head(agg3.reshape(-1, n, _COLW), z2, c3_root, c3_bias,
                 fc1_w, fc1_b, fc2_w, fc2_b, fc3_w, fc3_b, tn=1000)
    return y.reshape(-1)


# speedup vs baseline: 1.0214x; 1.0110x over previous
import functools

import jax
import jax.numpy as jnp
from jax import lax
from jax.experimental import pallas as pl
from jax.experimental.pallas import tpu as pltpu
from jax.experimental.pallas import tpu_sc as plsc

_NCORES = 2
_NSUB = 16


def _elu(v):
    return jnp.where(v > 0, v, jnp.exp(v) - 1.0)


def _sc_gather(table, idx):
    np_, d = table.shape
    e = idx.shape[0]
    nw = _NCORES * _NSUB
    bpw = e // nw

    mesh = plsc.VectorSubcoreMesh(core_axis_name="c", subcore_axis_name="s")

    @functools.partial(
        pl.kernel,
        out_type=jax.ShapeDtypeStruct((e, d), jnp.float32),
        mesh=mesh,
        scratch_types=[
            pltpu.VMEM((bpw,), jnp.int32),
            pltpu.VMEM((bpw, d), jnp.float32),
            pltpu.SemaphoreType.DMA,
        ],
    )
    def k(table_hbm, idx_hbm, out_hbm, idx_v, rows_v, sem):
        wid = lax.axis_index("s") * _NCORES + lax.axis_index("c")
        base = wid * bpw
        pltpu.sync_copy(idx_hbm.at[pl.ds(base, bpw)], idx_v)
        pltpu.async_copy(table_hbm.at[idx_v], rows_v, sem).wait()
        pltpu.sync_copy(rows_v, out_hbm.at[pl.ds(base, bpw)])

    return k(table, idx)


_COLW = 8


_ECH = 128


def _sc_scatter_cols(msg, zeros_flat, dst1d):
    e, out = msg.shape
    na = out // _COLW
    nf = zeros_flat.shape[0]

    mesh = plsc.VectorSubcoreMesh(core_axis_name="c", subcore_axis_name="s")

    @functools.partial(
        pl.kernel,
        out_type=jax.ShapeDtypeStruct((na, nf, 128), jnp.float32),
        mesh=mesh,
        compiler_params=pltpu.CompilerParams(needs_layout_passes=False),
        scratch_types=[
            pltpu.VMEM((e,), jnp.int32),
            pltpu.VMEM((_ECH, 128), jnp.float32),
            pltpu.VMEM((_ECH, 128), jnp.float32),
            pltpu.VMEM((nf, 128), jnp.float32),
            pltpu.SemaphoreType.DMA,
            pltpu.SemaphoreType.DMA,
        ],
    )
    def k(msg_hbm, zeros_hbm, dst_hbm, agg_hbm, idx_v, chunk_a, chunk_b,
          acc, sem_a, sem_b):
        w = lax.axis_index("s") * _NCORES + lax.axis_index("c")

        @pl.when(w < na)
        def _():
            pltpu.sync_copy(zeros_hbm, acc)
            pltpu.sync_copy(dst_hbm, idx_v)
            lanes = lax.iota(jnp.int32, 16)
            lo = lanes < _COLW
            half = (w // _NSUB) * 128
            wcol = w * _COLW - half
            colsel = wcol + (lanes & (_COLW - 1))
            bufs = ((chunk_a, sem_a), (chunk_b, sem_b))
            nch = e // _ECH

            def start(ch):
                buf, sem = bufs[ch % 2]
                return pltpu.async_copy(
                    msg_hbm.at[pl.ds(ch * _ECH, _ECH), pl.ds(half, 128)],
                    buf, sem)

            cp = start(0)
            for ch in range(nch):
                cp.wait()
                if ch + 1 < nch:
                    cp = start(ch + 1)
                chunk_v = bufs[ch % 2][0]

                def body(i, _):
                    base = 8 * i
                    loaded = []
                    for u in range(4):
                        lepair = base + 2 * u + (lanes >> 3)
                        rows = plsc.load_gather(idx_v, [ch * _ECH + lepair])
                        vals = plsc.load_gather(chunk_v, [lepair, colsel])
                        aflat = rows * _COLW + (lanes & (_COLW - 1))
                        loaded.append((aflat >> 7, aflat & 127, vals))
                    for ar, al, vals in loaded:
                        plsc.addupdate_scatter(acc, [ar, al], vals, mask=lo)
                        plsc.addupdate_scatter(acc, [ar, al], vals, mask=~lo)
                    return 0

                lax.fori_loop(0, _ECH // 8, body, 0)
            pltpu.sync_copy(acc, agg_hbm.at[w])

    return k(msg, zeros_flat, dst1d)


def _tc_messages(xg, eattr, w1, b1, w2r, b2r, *, ci, sub, te, apply_elu,
                 lci=None):
    e, in_pad = xg.shape
    hid = w1.shape[1]
    out = w2r.shape[1]
    if lci is None:
        lci = ci
    kc = lci * hid
    ic_n = w2r.shape[0] // kc
    ea_d = eattr.shape[1]

    def body(eattr_ref, xg_ref, w1_ref, b1_ref, w2r_ref, b2r_ref, acc_ref):
        ic = pl.program_id(0)
        et_i = pl.program_id(1)
        h = jnp.maximum(
            jnp.dot(eattr_ref[...], w1_ref[...],
                    preferred_element_type=jnp.float32) + b1_ref[...], 0.0)
        xc = xg_ref[...][:, :lci]
        if apply_elu:
            xc = _elu(xc)
        contrib = jnp.dot(xc, b2r_ref[...], preferred_element_type=jnp.float32)
        hb = h.astype(jnp.bfloat16)
        xb = xc.astype(jnp.bfloat16)
        for i in range(lci):
            pb = xb[:, i:i + 1] * hb
            contrib = contrib + jnp.dot(
                pb, w2r_ref[i * hid:(i + 1) * hid, :],
                preferred_element_type=jnp.float32)
        rows = pl.ds(et_i * te, te)

        @pl.when(ic == 0)
        def _():
            acc_ref[rows, :] = contrib

        @pl.when(ic != 0)
        def _():
            acc_ref[rows, :] = acc_ref[rows, :] + contrib

    return pl.pallas_call(
        body,
        grid=(ic_n, e // te),
        in_specs=[
            pl.BlockSpec((te, ea_d), lambda ic, et_i: (et_i, 0)),
            pl.BlockSpec((te, ci), lambda ic, et_i: (et_i, ic)),
            pl.BlockSpec(w1.shape, lambda ic, et_i: (0, 0)),
            pl.BlockSpec((1, hid), lambda ic, et_i: (0, 0)),
            pl.BlockSpec((kc, out), lambda ic, et_i: (ic, 0)),
            pl.BlockSpec((lci, out), lambda ic, et_i: (ic, 0)),
        ],
        out_specs=pl.BlockSpec((e, out), lambda ic, et_i: (0, 0)),
        out_shape=jax.ShapeDtypeStruct((e, out), jnp.float32),
    )(eattr, xg, w1, b1.reshape(1, hid), w2r, b2r)


def _tc_combine(agg_flat, x, rootw, bias, *, tn, apply_elu):
    nw, n, colw = agg_flat.shape
    cin = x.shape[1]
    out = rootw.shape[1]

    def body(a_ref, x_ref, w_ref, b_ref, o_ref):
        xt = x_ref[...]
        if apply_elu:
            xt = _elu(xt)
        agg = jnp.concatenate([a_ref[i] for i in range(nw)], axis=1)
        o_ref[...] = agg + jnp.dot(xt, w_ref[...],
                                   preferred_element_type=jnp.float32) + b_ref[...]

    return pl.pallas_call(
        body,
        grid=(n // tn,),
        in_specs=[
            pl.BlockSpec((nw, tn, colw), lambda i: (0, i, 0)),
            pl.BlockSpec((tn, cin), lambda i: (i, 0)),
            pl.BlockSpec(rootw.shape, lambda i: (0, 0)),
            pl.BlockSpec((1, out), lambda i: (0, 0)),
        ],
        out_specs=pl.BlockSpec((tn, out), lambda i: (i, 0)),
        out_shape=jax.ShapeDtypeStruct((n, out), jnp.float32),
    )(agg_flat, x, rootw, bias.reshape(1, out))


def _tc_head(agg_sh, x, rootw, rbias, fc1_w, fc1_b, fc2_w, fc2_b,
             fc3_w, fc3_b, *, tn):
    nw, n, colw = agg_sh.shape
    cin = x.shape[1]
    out = rootw.shape[1]

    def body(a_ref, x_ref, rw_ref, rb_ref, w1_ref, b1_ref, w2_ref, b2_ref,
             w3_ref, b3_ref, o_ref):
        agg = jnp.concatenate([a_ref[i] for i in range(nw)], axis=1)
        z = agg + jnp.dot(_elu(x_ref[...]), rw_ref[...],
                          preferred_element_type=jnp.float32) + rb_ref[...]
        h = _elu(z)
        h = _elu(jnp.dot(h, w1_ref[...],
                         preferred_element_type=jnp.float32) + b1_ref[...])
        h = _elu(jnp.dot(h, w2_ref[...],
                         preferred_element_type=jnp.float32) + b2_ref[...])
        o_ref[...] = jnp.dot(h, w3_ref[...],
                             preferred_element_type=jnp.float32) + b3_ref[...]

    return pl.pallas_call(
        body,
        grid=(n // tn,),
        in_specs=[
            pl.BlockSpec((nw, tn, colw), lambda i: (0, i, 0)),
            pl.BlockSpec((tn, cin), lambda i: (i, 0)),
            pl.BlockSpec(rootw.shape, lambda i: (0, 0)),
            pl.BlockSpec((1, out), lambda i: (0, 0)),
            pl.BlockSpec(fc1_w.shape, lambda i: (0, 0)),
            pl.BlockSpec((1, fc1_w.shape[1]), lambda i: (0, 0)),
            pl.BlockSpec(fc2_w.shape, lambda i: (0, 0)),
            pl.BlockSpec((1, fc2_w.shape[1]), lambda i: (0, 0)),
            pl.BlockSpec(fc3_w.shape, lambda i: (0, 0)),
            pl.BlockSpec((1, fc3_w.shape[1]), lambda i: (0, 0)),
        ],
        out_specs=pl.BlockSpec((tn, 1), lambda i: (i, 0)),
        out_shape=jax.ShapeDtypeStruct((n, 1), jnp.float32),
    )(agg_sh, x, rootw, rbias.reshape(1, out), fc1_w, fc1_b.reshape(1, -1),
      fc2_w, fc2_b.reshape(1, -1), fc3_w, fc3_b.reshape(1, -1))


def _prep_w2(w2, b2, hid, cin, cout, in_pad):
    w2r = w2.reshape(hid, cin, cout).transpose(1, 0, 2)
    w2r = jnp.pad(w2r, ((0, in_pad - cin), (0, 0), (0, 0)))
    b2r = jnp.pad(b2.reshape(cin, cout), ((0, in_pad - cin), (0, 0)))
    return w2r.reshape(in_pad * hid, cout).astype(jnp.bfloat16), b2r


def kernel(x, edge_index, edge_attr,
           c1_w1, c1_b1, c1_w2, c1_b2, c1_root, c1_bias,
           c2_w1, c2_b1, c2_w2, c2_b2, c2_root, c2_bias,
           c3_w1, c3_b1, c3_w2, c3_b2, c3_root, c3_bias,
           fc1_w, fc1_b, fc2_w, fc2_b, fc3_w, fc3_b):
    e = edge_index.shape[1]
    src = edge_index[0]
    dst1d = edge_index[1]
    n = x.shape[0]
    zeros_flat = jnp.zeros((n * _COLW // 128, 128), jnp.float32)


    in1 = x.shape[1]
    in1_pad = 48
    x_pad = jnp.pad(x, ((0, 0), (0, 128 - in1)))
    w2r1, b2r1 = _prep_w2(c1_w2, c1_b2, 512, in1, 128, in1_pad)
    root1 = jnp.pad(c1_root, ((0, 128 - in1), (0, 0)))

    xg1 = _sc_gather(x_pad, src)
    msg1 = _tc_messages(xg1, edge_attr, c1_w1, c1_b1, w2r1, b2r1,
                        ci=128, lci=48, sub=8, te=512, apply_elu=False)
    agg1 = _sc_scatter_cols(msg1, zeros_flat, dst1d)
    z1 = _tc_combine(agg1.reshape(-1, n, _COLW), x_pad, root1, c1_bias,
                     tn=1000, apply_elu=False)

    w2r2, b2r2 = _prep_w2(c2_w2, c2_b2, 128, 128, 256, 128)
    xg2 = _sc_gather(z1, src)
    msg2 = _tc_messages(xg2, edge_attr, c2_w1, c2_b1, w2r2, b2r2,
                        ci=128, sub=32, te=512, apply_elu=True)
    agg2 = _sc_scatter_cols(msg2, zeros_flat, dst1d)
    z2 = _tc_combine(agg2.reshape(-1, n, _COLW), z1, c2_root, c2_bias,
                     tn=1000, apply_elu=True)

    w2r3, b2r3 = _prep_w2(c3_w2, c3_b2, 128, 256, 256, 256)
    xg3 = _sc_gather(z2, src)
    msg3 = _tc_messages(xg3, edge_attr, c3_w1, c3_b1, w2r3, b2r3,
                        ci=128, sub=32, te=512, apply_elu=True)
    agg3 = _sc_scatter_cols(msg3, zeros_flat, dst1d)

    y = _tc_head(agg3.reshape(-1, n, _COLW), z2, c3_root, c3_bias,
                 fc1_w, fc1_b, fc2_w, fc2_b, fc3_w, fc3_b, tn=1000)
    return y.reshape(-1)


# speedup vs baseline: 1.0242x; 1.0028x over previous
import functools

import jax
import jax.numpy as jnp
from jax import lax
from jax.experimental import pallas as pl
from jax.experimental.pallas import tpu as pltpu
from jax.experimental.pallas import tpu_sc as plsc

_NCORES = 2
_NSUB = 16


def _elu(v):
    return jnp.where(v > 0, v, jnp.exp(v) - 1.0)


def _sc_gather(table, idx):
    np_, d = table.shape
    e = idx.shape[0]
    nw = _NCORES * _NSUB
    bpw = e // nw

    mesh = plsc.VectorSubcoreMesh(core_axis_name="c", subcore_axis_name="s")

    @functools.partial(
        pl.kernel,
        out_type=jax.ShapeDtypeStruct((e, d), jnp.float32),
        mesh=mesh,
        scratch_types=[
            pltpu.VMEM((bpw,), jnp.int32),
            pltpu.VMEM((bpw, d), jnp.float32),
            pltpu.SemaphoreType.DMA,
        ],
    )
    def k(table_hbm, idx_hbm, out_hbm, idx_v, rows_v, sem):
        wid = lax.axis_index("s") * _NCORES + lax.axis_index("c")
        base = wid * bpw
        pltpu.sync_copy(idx_hbm.at[pl.ds(base, bpw)], idx_v)
        pltpu.async_copy(table_hbm.at[idx_v], rows_v, sem).wait()
        pltpu.sync_copy(rows_v, out_hbm.at[pl.ds(base, bpw)])

    return k(table, idx)


_COLW = 8


_ECH = 128


def _sc_scatter_cols(msg, zeros_flat, dst1d):
    e, out = msg.shape
    na = out // _COLW
    nf = zeros_flat.shape[0]

    mesh = plsc.VectorSubcoreMesh(core_axis_name="c", subcore_axis_name="s")

    @functools.partial(
        pl.kernel,
        out_type=jax.ShapeDtypeStruct((na, nf, 128), jnp.float32),
        mesh=mesh,
        compiler_params=pltpu.CompilerParams(needs_layout_passes=False),
        scratch_types=[
            pltpu.VMEM((e,), jnp.int32),
            pltpu.VMEM((_ECH, 128), jnp.float32),
            pltpu.VMEM((_ECH, 128), jnp.float32),
            pltpu.VMEM((nf, 128), jnp.float32),
            pltpu.SemaphoreType.DMA,
            pltpu.SemaphoreType.DMA,
        ],
    )
    def k(msg_hbm, zeros_hbm, dst_hbm, agg_hbm, idx_v, chunk_a, chunk_b,
          acc, sem_a, sem_b):
        w = lax.axis_index("s") * _NCORES + lax.axis_index("c")

        @pl.when(w < na)
        def _():
            pltpu.sync_copy(zeros_hbm, acc)
            pltpu.sync_copy(dst_hbm, idx_v)
            lanes = lax.iota(jnp.int32, 16)
            lo = lanes < _COLW
            half = (w // _NSUB) * 128
            wcol = w * _COLW - half
            colsel = wcol + (lanes & (_COLW - 1))
            bufs = ((chunk_a, sem_a), (chunk_b, sem_b))
            nch = e // _ECH

            def start(ch):
                buf, sem = bufs[ch % 2]
                return pltpu.async_copy(
                    msg_hbm.at[pl.ds(ch * _ECH, _ECH), pl.ds(half, 128)],
                    buf, sem)

            cp = start(0)
            for ch in range(nch):
                cp.wait()
                if ch + 1 < nch:
                    cp = start(ch + 1)
                chunk_v = bufs[ch % 2][0]

                def body(i, _):
                    base = 8 * i
                    loaded = []
                    for u in range(4):
                        lepair = base + 2 * u + (lanes >> 3)
                        rows = plsc.load_gather(idx_v, [ch * _ECH + lepair])
                        vals = plsc.load_gather(chunk_v, [lepair, colsel])
                        aflat = rows * _COLW + (lanes & (_COLW - 1))
                        loaded.append((aflat >> 7, aflat & 127, vals))
                    for ar, al, vals in loaded:
                        plsc.addupdate_scatter(acc, [ar, al], vals, mask=lo)
                        plsc.addupdate_scatter(acc, [ar, al], vals, mask=~lo)
                    return 0

                lax.fori_loop(0, _ECH // 8, body, 0)
            pltpu.sync_copy(acc, agg_hbm.at[w])

    return k(msg, zeros_flat, dst1d)


def _tc_messages(xg, eattr, w1, b1, w2r, b2r, *, ci, sub, te, apply_elu,
                 lci=None):
    e, in_pad = xg.shape
    hid = w1.shape[1]
    out = w2r.shape[1]
    if lci is None:
        lci = ci
    kc = lci * hid
    ic_n = w2r.shape[0] // kc
    ea_d = eattr.shape[1]

    def body(eattr_ref, xg_ref, w1_ref, b1_ref, w2r_ref, b2r_ref, acc_ref):
        ic = pl.program_id(0)
        et_i = pl.program_id(1)
        h = jnp.maximum(
            jnp.dot(eattr_ref[...], w1_ref[...],
                    preferred_element_type=jnp.float32) + b1_ref[...], 0.0)
        xc = xg_ref[...][:, :lci]
        if apply_elu:
            xc = _elu(xc)
        contrib = jnp.dot(xc, b2r_ref[...], preferred_element_type=jnp.float32)
        hb = h.astype(jnp.bfloat16)
        xb = xc.astype(jnp.bfloat16)
        for i in range(lci):
            pb = xb[:, i:i + 1] * hb
            contrib = contrib + jnp.dot(
                pb, w2r_ref[i * hid:(i + 1) * hid, :],
                preferred_element_type=jnp.float32)
        rows = pl.ds(et_i * te, te)

        @pl.when(ic == 0)
        def _():
            acc_ref[rows, :] = contrib

        @pl.when(ic != 0)
        def _():
            acc_ref[rows, :] = acc_ref[rows, :] + contrib

    return pl.pallas_call(
        body,
        grid=(ic_n, e // te),
        in_specs=[
            pl.BlockSpec((te, ea_d), lambda ic, et_i: (et_i, 0)),
            pl.BlockSpec((te, ci), lambda ic, et_i: (et_i, ic)),
            pl.BlockSpec(w1.shape, lambda ic, et_i: (0, 0)),
            pl.BlockSpec((1, hid), lambda ic, et_i: (0, 0)),
            pl.BlockSpec((kc, out), lambda ic, et_i: (ic, 0)),
            pl.BlockSpec((lci, out), lambda ic, et_i: (ic, 0)),
        ],
        out_specs=pl.BlockSpec((e, out), lambda ic, et_i: (0, 0)),
        out_shape=jax.ShapeDtypeStruct((e, out), jnp.float32),
    )(eattr, xg, w1, b1.reshape(1, hid), w2r, b2r)


def _tc_combine(agg_flat, x, rootw, bias, *, tn, apply_elu):
    nw, n, colw = agg_flat.shape
    cin = x.shape[1]
    out = rootw.shape[1]

    def body(a_ref, x_ref, w_ref, b_ref, o_ref):
        xt = x_ref[...]
        if apply_elu:
            xt = _elu(xt)
        agg = jnp.concatenate([a_ref[i] for i in range(nw)], axis=1)
        o_ref[...] = agg + jnp.dot(xt, w_ref[...],
                                   preferred_element_type=jnp.float32) + b_ref[...]

    return pl.pallas_call(
        body,
        grid=(n // tn,),
        in_specs=[
            pl.BlockSpec((nw, tn, colw), lambda i: (0, i, 0)),
            pl.BlockSpec((tn, cin), lambda i: (i, 0)),
            pl.BlockSpec(rootw.shape, lambda i: (0, 0)),
            pl.BlockSpec((1, out), lambda i: (0, 0)),
        ],
        out_specs=pl.BlockSpec((tn, out), lambda i: (i, 0)),
        out_shape=jax.ShapeDtypeStruct((n, out), jnp.float32),
    )(agg_flat, x, rootw, bias.reshape(1, out))


def _tc_head(agg_sh, x, rootw, rbias, fc1_w, fc1_b, fc2_w, fc2_b,
             fc3_w, fc3_b, *, tn):
    nw, n, colw = agg_sh.shape
    cin = x.shape[1]
    out = rootw.shape[1]

    def body(a_ref, x_ref, rw_ref, rb_ref, w1_ref, b1_ref, w2_ref, b2_ref,
             w3_ref, b3_ref, o_ref):
        agg = jnp.concatenate([a_ref[i] for i in range(nw)], axis=1)
        z = agg + jnp.dot(_elu(x_ref[...]), rw_ref[...],
                          preferred_element_type=jnp.float32) + rb_ref[...]
        h = _elu(z)
        h = _elu(jnp.dot(h, w1_ref[...],
                         preferred_element_type=jnp.float32) + b1_ref[...])
        h = _elu(jnp.dot(h, w2_ref[...],
                         preferred_element_type=jnp.float32) + b2_ref[...])
        o_ref[...] = jnp.dot(h, w3_ref[...],
                             preferred_element_type=jnp.float32) + b3_ref[...]

    return pl.pallas_call(
        body,
        grid=(n // tn,),
        in_specs=[
            pl.BlockSpec((nw, tn, colw), lambda i: (0, i, 0)),
            pl.BlockSpec((tn, cin), lambda i: (i, 0)),
            pl.BlockSpec(rootw.shape, lambda i: (0, 0)),
            pl.BlockSpec((1, out), lambda i: (0, 0)),
            pl.BlockSpec(fc1_w.shape, lambda i: (0, 0)),
            pl.BlockSpec((1, fc1_w.shape[1]), lambda i: (0, 0)),
            pl.BlockSpec(fc2_w.shape, lambda i: (0, 0)),
            pl.BlockSpec((1, fc2_w.shape[1]), lambda i: (0, 0)),
            pl.BlockSpec(fc3_w.shape, lambda i: (0, 0)),
            pl.BlockSpec((1, fc3_w.shape[1]), lambda i: (0, 0)),
        ],
        out_specs=pl.BlockSpec((tn, 1), lambda i: (i, 0)),
        out_shape=jax.ShapeDtypeStruct((n, 1), jnp.float32),
    )(agg_sh, x, rootw, rbias.reshape(1, out), fc1_w, fc1_b.reshape(1, -1),
      fc2_w, fc2_b.reshape(1, -1), fc3_w, fc3_b.reshape(1, -1))


def _prep_w2(w2, b2, hid, cin, cout, in_pad):
    w2r = w2.reshape(hid, cin, cout).transpose(1, 0, 2)
    w2r = jnp.pad(w2r, ((0, in_pad - cin), (0, 0), (0, 0)))
    b2r = jnp.pad(b2.reshape(cin, cout), ((0, in_pad - cin), (0, 0)))
    return w2r.reshape(in_pad * hid, cout).astype(jnp.bfloat16), b2r


def kernel(x, edge_index, edge_attr,
           c1_w1, c1_b1, c1_w2, c1_b2, c1_root, c1_bias,
           c2_w1, c2_b1, c2_w2, c2_b2, c2_root, c2_bias,
           c3_w1, c3_b1, c3_w2, c3_b2, c3_root, c3_bias,
           fc1_w, fc1_b, fc2_w, fc2_b, fc3_w, fc3_b):
    e = edge_index.shape[1]
    src = edge_index[0]
    dst1d = edge_index[1]
    n = x.shape[0]
    zeros_flat = jnp.zeros((n * _COLW // 128, 128), jnp.float32)


    in1 = x.shape[1]
    in1_pad = 48
    x_pad = jnp.pad(x, ((0, 0), (0, 128 - in1)))
    w2r1, b2r1 = _prep_w2(c1_w2, c1_b2, 512, in1, 128, in1_pad)
    root1 = jnp.pad(c1_root, ((0, 128 - in1), (0, 0)))

    xg1 = _sc_gather(x_pad, src)
    msg1 = _tc_messages(xg1, edge_attr, c1_w1, c1_b1, w2r1, b2r1,
                        ci=128, lci=48, sub=8, te=1024, apply_elu=False)
    agg1 = _sc_scatter_cols(msg1, zeros_flat, dst1d)
    z1 = _tc_combine(agg1.reshape(-1, n, _COLW), x_pad, root1, c1_bias,
                     tn=1000, apply_elu=False)

    w2r2, b2r2 = _prep_w2(c2_w2, c2_b2, 128, 128, 256, 128)
    xg2 = _sc_gather(z1, src)
    msg2 = _tc_messages(xg2, edge_attr, c2_w1, c2_b1, w2r2, b2r2,
                        ci=128, sub=32, te=1024, apply_elu=True)
    agg2 = _sc_scatter_cols(msg2, zeros_flat, dst1d)
    z2 = _tc_combine(agg2.reshape(-1, n, _COLW), z1, c2_root, c2_bias,
                     tn=1000, apply_elu=True)

    w2r3, b2r3 = _prep_w2(c3_w2, c3_b2, 128, 256, 256, 256)
    xg3 = _sc_gather(z2, src)
    msg3 = _tc_messages(xg3, edge_attr, c3_w1, c3_b1, w2r3, b2r3,
                        ci=128, sub=32, te=1024, apply_elu=True)
    agg3 = _sc_scatter_cols(msg3, zeros_flat, dst1d)

    y = _tc_head(agg3.reshape(-1, n, _COLW), z2, c3_root, c3_bias,
                 fc1_w, fc1_b, fc2_w, fc2_b, fc3_w, fc3_b, tn=1000)
    return y.reshape(-1)
